# Initial kernel scaffold; baseline (speedup 1.0000x reference)
import jax
import jax.numpy as jnp
from jax.experimental import pallas as pl


def kernel(x, W, b):
    raise NotImplementedError("write your pallas kernel here")


# speedup vs baseline: 7.2578x; 7.2578x over previous
import functools

import jax
import jax.numpy as jnp
from jax.experimental import pallas as pl
from jax.experimental.pallas import tpu as pltpu

K = 20
BN = 256


def _prep_body(x_ref, w1_ref, wd_ref, bias_ref, y_ref, z_ref):
    x = x_ref[0]
    y_ref[0] = jnp.dot(x, w1_ref[...], preferred_element_type=jnp.float32)
    z_ref[0] = (
        jnp.dot(x, wd_ref[...], preferred_element_type=jnp.float32) + bias_ref[...]
    )


def _edge_body(xr_ref, xf_ref, y_ref, z_ref, out_ref):
    xr = xr_ref[0]
    xf = xf_ref[0]
    n = xf.shape[0]
    sq_r = jnp.sum(xr * xr, axis=1, keepdims=True)
    sq_f = jnp.sum(xf * xf, axis=1, keepdims=True)
    d = (
        sq_r
        + sq_f.reshape(1, n)
        - 2.0
        * jax.lax.dot_general(xr, xf, (((1,), (1,)), ((), ())),
                              preferred_element_type=jnp.float32)
    )
    y = y_ref[0]
    iota = jax.lax.broadcasted_iota(jnp.int32, d.shape, 1)

    def body(_, carry):
        d, acc = carry
        m = jnp.min(d, axis=1, keepdims=True)
        jmin = jnp.min(jnp.where(d == m, iota, n), axis=1, keepdims=True)
        onehot = iota == jmin
        sel = jnp.dot(onehot.astype(jnp.float32), y,
                      preferred_element_type=jnp.float32)
        acc = jnp.maximum(acc, sel)
        d = jnp.where(onehot, jnp.inf, d)
        return d, acc

    acc0 = jnp.full((xr.shape[0], y.shape[1]), -jnp.inf, jnp.float32)
    _, acc = jax.lax.fori_loop(0, K, body, (d, acc0))
    h = acc + z_ref[0]
    out_ref[0] = jnp.where(h > 0, h, 0.2 * h)


def kernel(x, W, b):
    B, N, C = x.shape
    w1 = W[:C]
    wd = W[C:] - W[:C]

    y, z = pl.pallas_call(
        _prep_body,
        grid=(B,),
        in_specs=[
            pl.BlockSpec((1, N, C), lambda i: (i, 0, 0)),
            pl.BlockSpec((C, C), lambda i: (0, 0)),
            pl.BlockSpec((C, C), lambda i: (0, 0)),
            pl.BlockSpec((C,), lambda i: (0,)),
        ],
        out_specs=[
            pl.BlockSpec((1, N, C), lambda i: (i, 0, 0)),
            pl.BlockSpec((1, N, C), lambda i: (i, 0, 0)),
        ],
        out_shape=[
            jax.ShapeDtypeStruct((B, N, C), jnp.float32),
            jax.ShapeDtypeStruct((B, N, C), jnp.float32),
        ],
    )(x, w1, wd, b)

    out = pl.pallas_call(
        _edge_body,
        grid=(B, N // BN),
        in_specs=[
            pl.BlockSpec((1, BN, C), lambda i, r: (i, r, 0)),
            pl.BlockSpec((1, N, C), lambda i, r: (i, 0, 0)),
            pl.BlockSpec((1, N, C), lambda i, r: (i, 0, 0)),
            pl.BlockSpec((1, BN, C), lambda i, r: (i, r, 0)),
        ],
        out_specs=pl.BlockSpec((1, BN, C), lambda i, r: (i, r, 0)),
        out_shape=jax.ShapeDtypeStruct((B, N, C), jnp.float32),
    )(x, x, y, z)
    return out


# speedup vs baseline: 8.3373x; 1.1487x over previous
import functools

import jax
import jax.numpy as jnp
from jax import lax
from jax.experimental import pallas as pl
from jax.experimental.pallas import tpu as pltpu
from jax.experimental.pallas import tpu_sc as plsc

K = 20
KP = 24
BN = 256
CP = 16
NSUB = 128


def _prep_body(x_ref, w1_ref, wd_ref, bias_ref, y_ref, z_ref):
    x = x_ref[0]
    y_ref[0] = jnp.dot(x, w1_ref[...], preferred_element_type=jnp.float32)
    z_ref[0] = (
        jnp.dot(x, wd_ref[...], preferred_element_type=jnp.float32) + bias_ref[...]
    )


def _topk_body(xr_ref, xf_ref, idx_ref):
    b = pl.program_id(0)
    xr = xr_ref[0]
    xf = xf_ref[0]
    n = xf.shape[0]
    sq_r = jnp.sum(xr * xr, axis=1, keepdims=True)
    sq_f = jnp.sum(xf * xf, axis=1, keepdims=True)
    d = (
        sq_r
        + sq_f.reshape(1, n)
        - 2.0
        * lax.dot_general(xr, xf, (((1,), (1,)), ((), ())),
                          preferred_element_type=jnp.float32)
    )
    iota = lax.broadcasted_iota(jnp.int32, d.shape, 1)
    col_iota = lax.broadcasted_iota(jnp.int32, (d.shape[0], KP), 1)

    def body(t, carry):
        d, idxacc = carry
        m = jnp.min(d, axis=1, keepdims=True)
        jmin = jnp.min(jnp.where(d == m, iota, n), axis=1, keepdims=True)
        idxacc = jnp.where(col_iota >= t, jmin, idxacc)
        d = jnp.where(iota == jmin, jnp.inf, d)
        return d, idxacc

    idx0 = jnp.zeros((d.shape[0], KP), jnp.int32)
    _, idxacc = lax.fori_loop(0, K, body, (d, idx0))
    idx_ref[0] = idxacc + b * n


def _sc_gather_max(P, C):
    NC, NS = 2, 16
    NW = NC * NS
    PPW = P // NW
    NCH = PPW // CP
    assert CP * KP == 3 * NSUB

    mesh = plsc.VectorSubcoreMesh(core_axis_name="c", subcore_axis_name="s")

    @functools.partial(
        pl.kernel,
        mesh=mesh,
        out_type=jax.ShapeDtypeStruct((P, C), jnp.float32),
        scratch_types=[
            pltpu.VMEM((CP * KP,), jnp.int32),
            pltpu.VMEM((CP * KP,), jnp.int32),
            pltpu.VMEM((CP * KP, C), jnp.float32),
            pltpu.VMEM((CP * KP, C), jnp.float32),
            pltpu.VMEM((CP, C), jnp.float32),
            pltpu.VMEM((CP, C), jnp.float32),
            pltpu.VMEM((CP, C), jnp.float32),
            pltpu.SemaphoreType.DMA,
            pltpu.SemaphoreType.DMA,
            pltpu.SemaphoreType.DMA,
            pltpu.SemaphoreType.DMA,
        ],
    )
    def gather_max(y_hbm, idx_hbm, z_hbm, out_hbm,
                   idx0, idx1, rows0, rows1, z0, z1, out_v,
                   gsem0, gsem1, zsem0, zsem1):
        c = lax.axis_index("c")
        s = lax.axis_index("s")
        wid = s * NC + c
        base = wid * PPW

        def prefetch(g, idxb, rowsb, zb, gsem, zsem):
            pbase = base + g * CP
            off = pl.multiple_of(pbase * KP, 8)
            pltpu.sync_copy(idx_hbm.at[pl.ds(off, CP * KP)], idxb)
            for t in range(3):
                pltpu.make_async_copy(
                    y_hbm.at[idxb.at[pl.ds(t * NSUB, NSUB)]],
                    rowsb.at[pl.ds(t * NSUB, NSUB)],
                    gsem,
                ).start()
            pltpu.make_async_copy(z_hbm.at[pl.ds(pbase, CP)], zb, zsem).start()

        def wait_chunk(g, idxb, rowsb, zb, gsem, zsem):
            pbase = base + g * CP
            for t in range(3):
                pltpu.make_async_copy(
                    y_hbm.at[idxb.at[pl.ds(t * NSUB, NSUB)]],
                    rowsb.at[pl.ds(t * NSUB, NSUB)],
                    gsem,
                ).wait()
            pltpu.make_async_copy(z_hbm.at[pl.ds(pbase, CP)], zb, zsem).wait()

        def compute(g, rowsb, zb):
            pbase = base + g * CP

            def pt(p, _):
                rbase = p * KP
                for cg in range(C // 16):
                    sl = pl.ds(cg * 16, 16)
                    acc = rowsb[rbase, sl]
                    for j in range(1, KP):
                        acc = jnp.maximum(acc, rowsb[rbase + j, sl])
                    h = acc + zb[p, sl]
                    out_v[p, sl] = jnp.where(h > 0, h, 0.2 * h)
                return 0

            lax.fori_loop(0, CP, pt, 0)
            pltpu.sync_copy(out_v, out_hbm.at[pl.ds(pbase, CP)])

        prefetch(0, idx0, rows0, z0, gsem0, zsem0)

        def super_body(hh, _):
            g0 = 2 * hh
            prefetch(g0 + 1, idx1, rows1, z1, gsem1, zsem1)
            wait_chunk(g0, idx0, rows0, z0, gsem0, zsem0)
            compute(g0, rows0, z0)

            @pl.when(hh < NCH // 2 - 1)
            def _():
                prefetch(g0 + 2, idx0, rows0, z0, gsem0, zsem0)

            wait_chunk(g0 + 1, idx1, rows1, z1, gsem1, zsem1)
            compute(g0 + 1, rows1, z1)
            return 0

        lax.fori_loop(0, NCH // 2, super_body, 0)

    return gather_max


def kernel(x, W, b):
    B, N, C = x.shape
    w1 = W[:C]
    wd = W[C:] - W[:C]

    y, z = pl.pallas_call(
        _prep_body,
        grid=(B,),
        in_specs=[
            pl.BlockSpec((1, N, C), lambda i: (i, 0, 0)),
            pl.BlockSpec((C, C), lambda i: (0, 0)),
            pl.BlockSpec((C, C), lambda i: (0, 0)),
            pl.BlockSpec((C,), lambda i: (0,)),
        ],
        out_specs=[
            pl.BlockSpec((1, N, C), lambda i: (i, 0, 0)),
            pl.BlockSpec((1, N, C), lambda i: (i, 0, 0)),
        ],
        out_shape=[
            jax.ShapeDtypeStruct((B, N, C), jnp.float32),
            jax.ShapeDtypeStruct((B, N, C), jnp.float32),
        ],
    )(x, w1, wd, b)

    idx = pl.pallas_call(
        _topk_body,
        grid=(B, N // BN),
        in_specs=[
            pl.BlockSpec((1, BN, C), lambda i, r: (i, r, 0)),
            pl.BlockSpec((1, N, C), lambda i, r: (i, 0, 0)),
        ],
        out_specs=pl.BlockSpec((1, BN, KP), lambda i, r: (i, r, 0)),
        out_shape=jax.ShapeDtypeStruct((B, N, KP), jnp.int32),
    )(x, x)

    P = B * N
    out = _sc_gather_max(P, C)(
        y.reshape(P, C), idx.reshape(P * KP), z.reshape(P, C)
    )
    return out.reshape(B, N, C)
